# baseline (speedup 1.0000x reference)
import jax
import jax.numpy as jnp
from jax import lax
from jax.experimental import pallas as pl
from jax.experimental.pallas import tpu as pltpu
from jax.experimental.pallas import tpu_sc as plsc

NUM_EMB = 1000000
DIM = 32
B1 = 16384
S = 50
B_TOTAL = B1 * S
NC, NS = 2, 16
NW = NC * NS
IDX_PER_GROUP = 128
GROUPS_PER_W = B_TOTAL // (NW * IDX_PER_GROUP)
G_PER_CHUNK = 8
CHUNK_ROWS = G_PER_CHUNK * IDX_PER_GROUP
N_CHUNKS = GROUPS_PER_W // G_PER_CHUNK
N_BLOCKS = B_TOTAL // 512


def _gather_body(idx_hbm, table_hbm, out_hbm, idx_v, buf0, buf1, gsem0, gsem1):
    c = lax.axis_index("c")
    s = lax.axis_index("s")
    wid = s * NC + c
    gbase = wid * GROUPS_PER_W
    bbase = wid * (GROUPS_PER_W // 4)

    pltpu.sync_copy(idx_hbm.at[pl.ds(gbase, GROUPS_PER_W)], idx_v)

    bufs = (buf0, buf1)
    gsems = (gsem0, gsem1)

    def fire(chunk, b):
        for j in range(G_PER_CHUNK):
            pltpu.async_copy(
                table_hbm.at[idx_v.at[chunk * G_PER_CHUNK + j]],
                bufs[b].at[pl.ds(j * IDX_PER_GROUP, IDX_PER_GROUP)],
                gsems[b],
            )

    def drain(b):
        pltpu.make_async_copy(
            table_hbm.at[pl.ds(0, CHUNK_ROWS)], bufs[b], gsems[b]
        ).wait()

    def scatter(chunk, b):
        for j in range(G_PER_CHUNK):
            blk = bbase + chunk * (G_PER_CHUNK // 4) + (j // 4)
            pltpu.sync_copy(
                bufs[b].at[pl.ds(j * IDX_PER_GROUP, IDX_PER_GROUP)],
                out_hbm.at[blk, :, j % 4, :],
            )

    fire(0, 0)
    fire(1, 1)

    def step(it, carry):
        chunk = it * 2
        for b in range(2):
            cc = chunk + b
            drain(b)
            scatter(cc, b)

            @pl.when(cc + 2 < N_CHUNKS)
            def _():
                fire(cc + 2, b)

        return carry

    lax.fori_loop(0, N_CHUNKS // 2, step, 0)
    drain(0)
    scatter(N_CHUNKS - 1, 0)


RB = 1024
NBB = B1 // (4 * RB)


def _tr_body(x_ref, o_ref):
    xt = jnp.transpose(x_ref[...])
    for q in range(4):
        for blk in range(RB // 128):
            o_ref[0, :, blk * 512 + q * 128:blk * 512 + (q + 1) * 128] = (
                xt[32 * q:32 * q + 32, blk * 128:(blk + 1) * 128]
            )


@jax.jit
def _emb_lookup(idx_flat, w128):
    weight = jax.lax.optimization_barrier(w128).reshape(NUM_EMB, DIM)
    mesh = plsc.VectorSubcoreMesh(
        core_axis_name="c", subcore_axis_name="s", num_cores=NC, num_subcores=NS
    )
    gather = pl.kernel(
        _gather_body,
        out_type=jax.ShapeDtypeStruct((N_BLOCKS, IDX_PER_GROUP, 4, DIM), jnp.float32),
        mesh=mesh,
        scratch_types=[
            pltpu.VMEM((GROUPS_PER_W, IDX_PER_GROUP), jnp.int32),
            pltpu.VMEM((CHUNK_ROWS, DIM), jnp.float32),
            pltpu.VMEM((CHUNK_ROWS, DIM), jnp.float32),
            pltpu.SemaphoreType.DMA,
            pltpu.SemaphoreType.DMA,
        ],
        compiler_params=pltpu.CompilerParams(use_tc_tiling_on_sc=False),
    )
    rows = gather(idx_flat, weight)

    rows_v = rows.reshape(B_TOTAL * DIM // 128, 128)

    out_t = pl.pallas_call(
        _tr_body,
        grid=(S, NBB),
        in_specs=[
            pl.BlockSpec((RB, 128), lambda si, bi: (si * NBB + bi, 0)),
        ],
        out_specs=pl.BlockSpec((1, DIM, 4 * RB), lambda si, bi: (si, 0, bi)),
        out_shape=jax.ShapeDtypeStruct((S, DIM, B1), jnp.float32),
    )(rows_v)

    return jnp.transpose(out_t, (2, 0, 1))


def kernel(input_, weight):
    idx_flat = (
        input_.astype(jnp.int32).T.reshape(B_TOTAL // IDX_PER_GROUP, IDX_PER_GROUP)
    )
    return _emb_lookup(idx_flat, weight.reshape(NUM_EMB // 4, 4 * DIM))


# speedup vs baseline: 1.0638x; 1.0638x over previous
import jax
import jax.numpy as jnp
from jax import lax
from jax.experimental import pallas as pl
from jax.experimental.pallas import tpu as pltpu
from jax.experimental.pallas import tpu_sc as plsc

NUM_EMB = 1000000
DIM = 32
B1 = 16384
S = 50
B_TOTAL = B1 * S
NC, NS = 2, 16
NW = NC * NS
IDX_PER_GROUP = 128
GROUPS_PER_W = B_TOTAL // (NW * IDX_PER_GROUP)
G_PER_CHUNK = 8
CHUNK_ROWS = G_PER_CHUNK * IDX_PER_GROUP
N_CHUNKS = GROUPS_PER_W // G_PER_CHUNK
N_BLOCKS = B_TOTAL // 512


def _gather_body(idx_hbm, table_hbm, out_hbm, idx_v, buf0, buf1, gsem0, gsem1):
    c = lax.axis_index("c")
    s = lax.axis_index("s")
    wid = s * NC + c
    gbase = wid * GROUPS_PER_W
    bbase = wid * (GROUPS_PER_W // 4)

    pltpu.sync_copy(idx_hbm.at[pl.ds(gbase, GROUPS_PER_W)], idx_v)

    bufs = (buf0, buf1)
    gsems = (gsem0, gsem1)

    def fire(chunk, b):
        for j in range(G_PER_CHUNK):
            pltpu.async_copy(
                table_hbm.at[idx_v.at[chunk * G_PER_CHUNK + j]],
                bufs[b].at[pl.ds(j * IDX_PER_GROUP, IDX_PER_GROUP)],
                gsems[b],
            )

    def drain(b):
        pltpu.make_async_copy(
            table_hbm.at[pl.ds(0, CHUNK_ROWS)], bufs[b], gsems[b]
        ).wait()

    def scatter(chunk, b):
        for j in range(G_PER_CHUNK):
            blk = bbase + chunk * (G_PER_CHUNK // 4) + (j // 4)
            pltpu.sync_copy(
                bufs[b].at[pl.ds(j * IDX_PER_GROUP, IDX_PER_GROUP)],
                out_hbm.at[blk, :, j % 4, :],
            )

    fire(0, 0)
    fire(1, 1)

    def step(it, carry):
        chunk = it * 2
        for b in range(2):
            cc = chunk + b
            drain(b)
            scatter(cc, b)

            @pl.when(cc + 2 < N_CHUNKS)
            def _():
                fire(cc + 2, b)

        return carry

    lax.fori_loop(0, N_CHUNKS // 2, step, 0)
    drain(0)
    scatter(N_CHUNKS - 1, 0)


RB = 2048
NBB = B1 // (4 * RB)


def _tr_body(x_ref, o_ref):
    xt = jnp.transpose(x_ref[...])
    for q in range(4):
        for blk in range(RB // 128):
            o_ref[0, :, blk * 512 + q * 128:blk * 512 + (q + 1) * 128] = (
                xt[32 * q:32 * q + 32, blk * 128:(blk + 1) * 128]
            )


@jax.jit
def _emb_lookup(idx_flat, w128):
    weight = jax.lax.optimization_barrier(w128).reshape(NUM_EMB, DIM)
    mesh = plsc.VectorSubcoreMesh(
        core_axis_name="c", subcore_axis_name="s", num_cores=NC, num_subcores=NS
    )
    gather = pl.kernel(
        _gather_body,
        out_type=jax.ShapeDtypeStruct((N_BLOCKS, IDX_PER_GROUP, 4, DIM), jnp.float32),
        mesh=mesh,
        scratch_types=[
            pltpu.VMEM((GROUPS_PER_W, IDX_PER_GROUP), jnp.int32),
            pltpu.VMEM((CHUNK_ROWS, DIM), jnp.float32),
            pltpu.VMEM((CHUNK_ROWS, DIM), jnp.float32),
            pltpu.SemaphoreType.DMA,
            pltpu.SemaphoreType.DMA,
        ],
        compiler_params=pltpu.CompilerParams(use_tc_tiling_on_sc=False),
    )
    rows = gather(idx_flat, weight)

    rows_v = rows.reshape(B_TOTAL * DIM // 128, 128)

    out_t = pl.pallas_call(
        _tr_body,
        grid=(S, NBB),
        in_specs=[
            pl.BlockSpec((RB, 128), lambda si, bi: (si * NBB + bi, 0)),
        ],
        out_specs=pl.BlockSpec((1, DIM, 4 * RB), lambda si, bi: (si, 0, bi)),
        out_shape=jax.ShapeDtypeStruct((S, DIM, B1), jnp.float32),
    )(rows_v)

    return jnp.transpose(out_t, (2, 0, 1))


def kernel(input_, weight):
    idx_flat = (
        input_.astype(jnp.int32).T.reshape(B_TOTAL // IDX_PER_GROUP, IDX_PER_GROUP)
    )
    return _emb_lookup(idx_flat, weight.reshape(NUM_EMB // 4, 4 * DIM))


# speedup vs baseline: 1.1198x; 1.0527x over previous
import jax
import jax.numpy as jnp
from jax import lax
from jax.experimental import pallas as pl
from jax.experimental.pallas import tpu as pltpu
from jax.experimental.pallas import tpu_sc as plsc

NUM_EMB = 1000000
DIM = 32
B1 = 16384
S = 50
B_TOTAL = B1 * S
NC, NS = 2, 16
NW = NC * NS
IDX_PER_GROUP = 128
GROUPS_PER_W = B_TOTAL // (NW * IDX_PER_GROUP)
G_PER_CHUNK = 8
CHUNK_ROWS = G_PER_CHUNK * IDX_PER_GROUP
N_CHUNKS = GROUPS_PER_W // G_PER_CHUNK
N_BLOCKS = B_TOTAL // 512


def _gather_body(idx_hbm, table_hbm, out_hbm, idx_v, buf0, buf1, gsem0, gsem1):
    c = lax.axis_index("c")
    s = lax.axis_index("s")
    wid = s * NC + c
    gbase = wid * GROUPS_PER_W
    bbase = wid * (GROUPS_PER_W // 4)

    pltpu.sync_copy(idx_hbm.at[pl.ds(gbase, GROUPS_PER_W)], idx_v)

    bufs = (buf0, buf1)
    gsems = (gsem0, gsem1)

    def fire(chunk, b):
        for j in range(G_PER_CHUNK):
            pltpu.async_copy(
                table_hbm.at[idx_v.at[chunk * G_PER_CHUNK + j]],
                bufs[b].at[pl.ds(j * IDX_PER_GROUP, IDX_PER_GROUP)],
                gsems[b],
            )

    def drain(b):
        pltpu.make_async_copy(
            table_hbm.at[pl.ds(0, CHUNK_ROWS)], bufs[b], gsems[b]
        ).wait()

    def scatter(chunk, b):
        for j in range(G_PER_CHUNK):
            blk = bbase + chunk * (G_PER_CHUNK // 4) + (j // 4)
            pltpu.sync_copy(
                bufs[b].at[pl.ds(j * IDX_PER_GROUP, IDX_PER_GROUP)],
                out_hbm.at[blk, :, j % 4, :],
            )

    fire(0, 0)
    fire(1, 1)

    def step(it, carry):
        chunk = it * 2
        for b in range(2):
            cc = chunk + b
            drain(b)
            scatter(cc, b)

            @pl.when(cc + 2 < N_CHUNKS)
            def _():
                fire(cc + 2, b)

        return carry

    lax.fori_loop(0, N_CHUNKS // 2, step, 0)
    drain(0)
    scatter(N_CHUNKS - 1, 0)


RB = 4096
NBB = B1 // (4 * RB)


def _tr_body(x_ref, o_ref):
    xt = jnp.transpose(x_ref[...])
    for q in range(4):
        for blk in range(RB // 128):
            o_ref[0, :, blk * 512 + q * 128:blk * 512 + (q + 1) * 128] = (
                xt[32 * q:32 * q + 32, blk * 128:(blk + 1) * 128]
            )


@jax.jit
def _emb_lookup(idx_flat, w128):
    weight = jax.lax.optimization_barrier(w128).reshape(NUM_EMB, DIM)
    mesh = plsc.VectorSubcoreMesh(
        core_axis_name="c", subcore_axis_name="s", num_cores=NC, num_subcores=NS
    )
    gather = pl.kernel(
        _gather_body,
        out_type=jax.ShapeDtypeStruct((N_BLOCKS, IDX_PER_GROUP, 4, DIM), jnp.float32),
        mesh=mesh,
        scratch_types=[
            pltpu.VMEM((GROUPS_PER_W, IDX_PER_GROUP), jnp.int32),
            pltpu.VMEM((CHUNK_ROWS, DIM), jnp.float32),
            pltpu.VMEM((CHUNK_ROWS, DIM), jnp.float32),
            pltpu.SemaphoreType.DMA,
            pltpu.SemaphoreType.DMA,
        ],
        compiler_params=pltpu.CompilerParams(use_tc_tiling_on_sc=False),
    )
    rows = gather(idx_flat, weight)

    rows_v = rows.reshape(B_TOTAL * DIM // 128, 128)

    out_t = pl.pallas_call(
        _tr_body,
        grid=(S, NBB),
        in_specs=[
            pl.BlockSpec((RB, 128), lambda si, bi: (si * NBB + bi, 0)),
        ],
        out_specs=pl.BlockSpec((1, DIM, 4 * RB), lambda si, bi: (si, 0, bi)),
        out_shape=jax.ShapeDtypeStruct((S, DIM, B1), jnp.float32),
    )(rows_v)

    return jnp.transpose(out_t, (2, 0, 1))


def kernel(input_, weight):
    idx_flat = (
        input_.astype(jnp.int32).T.reshape(B_TOTAL // IDX_PER_GROUP, IDX_PER_GROUP)
    )
    return _emb_lookup(idx_flat, weight.reshape(NUM_EMB // 4, 4 * DIM))


# speedup vs baseline: 1.7917x; 1.6000x over previous
import jax
import jax.numpy as jnp
from jax import lax
from jax.experimental import pallas as pl
from jax.experimental.pallas import tpu as pltpu
from jax.experimental.pallas import tpu_sc as plsc

NUM_EMB = 1000000
DIM = 32
B1 = 16384
S = 50
B_TOTAL = B1 * S
NC, NS = 2, 16
NW = NC * NS
IDX_PER_GROUP = 128
GROUPS_PER_W = B_TOTAL // (NW * IDX_PER_GROUP)
G_PER_CHUNK = 8
CHUNK_ROWS = G_PER_CHUNK * IDX_PER_GROUP
N_CHUNKS = GROUPS_PER_W // G_PER_CHUNK
N_BLOCKS = B_TOTAL // 512


def _gather_body(idx_hbm, table_hbm, out_hbm, idx_v, buf0, buf1, gsem0, gsem1):
    c = lax.axis_index("c")
    s = lax.axis_index("s")
    wid = s * NC + c
    gbase = wid * GROUPS_PER_W
    bbase = wid * (GROUPS_PER_W // 4)

    pltpu.sync_copy(idx_hbm.at[pl.ds(gbase, GROUPS_PER_W)], idx_v)

    bufs = (buf0, buf1)
    gsems = (gsem0, gsem1)

    def fire(chunk, b):
        for j in range(G_PER_CHUNK):
            pltpu.async_copy(
                table_hbm.at[idx_v.at[chunk * G_PER_CHUNK + j]],
                bufs[b].at[pl.ds(j * IDX_PER_GROUP, IDX_PER_GROUP)],
                gsems[b],
            )

    def drain(b):
        pltpu.make_async_copy(
            table_hbm.at[pl.ds(0, CHUNK_ROWS)], bufs[b], gsems[b]
        ).wait()

    def scatter(chunk, b):
        for j in range(G_PER_CHUNK):
            blk = bbase + chunk * (G_PER_CHUNK // 4) + (j // 4)
            pltpu.sync_copy(
                bufs[b].at[pl.ds(j * IDX_PER_GROUP, IDX_PER_GROUP)],
                out_hbm.at[blk, :, j % 4, :],
            )

    fire(0, 0)
    fire(1, 1)

    def step(it, carry):
        chunk = it * 2
        for b in range(2):
            cc = chunk + b
            drain(b)
            scatter(cc, b)

            @pl.when(cc + 2 < N_CHUNKS)
            def _():
                fire(cc + 2, b)

        return carry

    lax.fori_loop(0, N_CHUNKS // 2, step, 0)
    drain(0)
    scatter(N_CHUNKS - 1, 0)


RB = 4096
NBB = B1 // (4 * RB)


def _tr_body(x_ref, o_ref):
    xt = jnp.transpose(x_ref[...])
    for q in range(4):
        for blk in range(RB // 128):
            o_ref[0, :, blk * 512 + q * 128:blk * 512 + (q + 1) * 128] = (
                xt[32 * q:32 * q + 32, blk * 128:(blk + 1) * 128]
            )


C0 = 8192
NPB = -(-NUM_EMB // C0)


def _pack_body(x_ref, o_ref):
    o_ref[:, 0:DIM] = jnp.transpose(x_ref[...])


@jax.jit
def _emb_lookup(idx_flat, w_t):
    w_pad = pl.pallas_call(
        _pack_body,
        grid=(NPB,),
        in_specs=[pl.BlockSpec((DIM, C0), lambda i: (0, i))],
        out_specs=pl.BlockSpec((C0, 4 * DIM), lambda i: (i, 0)),
        out_shape=jax.ShapeDtypeStruct((NUM_EMB, 4 * DIM), jnp.float32),
    )(w_t)
    weight = w_pad.reshape(4 * NUM_EMB, DIM)
    mesh = plsc.VectorSubcoreMesh(
        core_axis_name="c", subcore_axis_name="s", num_cores=NC, num_subcores=NS
    )
    gather = pl.kernel(
        _gather_body,
        out_type=jax.ShapeDtypeStruct((N_BLOCKS, IDX_PER_GROUP, 4, DIM), jnp.float32),
        mesh=mesh,
        scratch_types=[
            pltpu.VMEM((GROUPS_PER_W, IDX_PER_GROUP), jnp.int32),
            pltpu.VMEM((CHUNK_ROWS, DIM), jnp.float32),
            pltpu.VMEM((CHUNK_ROWS, DIM), jnp.float32),
            pltpu.SemaphoreType.DMA,
            pltpu.SemaphoreType.DMA,
        ],
        compiler_params=pltpu.CompilerParams(use_tc_tiling_on_sc=False),
    )
    rows = gather(idx_flat, weight)

    rows_v = rows.reshape(B_TOTAL * DIM // 128, 128)

    out_t = pl.pallas_call(
        _tr_body,
        grid=(S, NBB),
        in_specs=[
            pl.BlockSpec((RB, 128), lambda si, bi: (si * NBB + bi, 0)),
        ],
        out_specs=pl.BlockSpec((1, DIM, 4 * RB), lambda si, bi: (si, 0, bi)),
        out_shape=jax.ShapeDtypeStruct((S, DIM, B1), jnp.float32),
    )(rows_v)

    return jnp.transpose(out_t, (2, 0, 1))


def kernel(input_, weight):
    idx_flat = (
        (input_.astype(jnp.int32) * 4).T
        .reshape(B_TOTAL // IDX_PER_GROUP, IDX_PER_GROUP)
    )
    return _emb_lookup(idx_flat, weight.T)


# speedup vs baseline: 1.9375x; 1.0813x over previous
import jax
import jax.numpy as jnp
from jax import lax
from jax.experimental import pallas as pl
from jax.experimental.pallas import tpu as pltpu
from jax.experimental.pallas import tpu_sc as plsc

NUM_EMB = 1000000
DIM = 32
B1 = 16384
S = 50
B_TOTAL = B1 * S
NC, NS = 2, 16
NW = NC * NS
IDX_PER_GROUP = 128
GROUPS_PER_W = B_TOTAL // (NW * IDX_PER_GROUP)
G_PER_CHUNK = 8
CHUNK_ROWS = G_PER_CHUNK * IDX_PER_GROUP
N_CHUNKS = GROUPS_PER_W // G_PER_CHUNK
N_BLOCKS = B_TOTAL // 512


def _gather_body(idx_hbm, table_hbm, out_hbm, idx_v, buf0, buf1, gsem0, gsem1):
    c = lax.axis_index("c")
    s = lax.axis_index("s")
    wid = s * NC + c
    gbase = wid * GROUPS_PER_W
    bbase = wid * (GROUPS_PER_W // 4)

    pltpu.sync_copy(idx_hbm.at[pl.ds(gbase, GROUPS_PER_W)], idx_v)

    bufs = (buf0, buf1)
    gsems = (gsem0, gsem1)

    def fire(chunk, b):
        for j in range(G_PER_CHUNK):
            pltpu.async_copy(
                table_hbm.at[idx_v.at[chunk * G_PER_CHUNK + j]],
                bufs[b].at[pl.ds(j * IDX_PER_GROUP, IDX_PER_GROUP)],
                gsems[b],
            )

    def drain(b):
        pltpu.make_async_copy(
            table_hbm.at[pl.ds(0, CHUNK_ROWS)], bufs[b], gsems[b]
        ).wait()

    def scatter(chunk, b):
        for j in range(G_PER_CHUNK):
            blk = bbase + chunk * (G_PER_CHUNK // 4) + (j // 4)
            pltpu.sync_copy(
                bufs[b].at[pl.ds(j * IDX_PER_GROUP, IDX_PER_GROUP)],
                out_hbm.at[blk, :, j % 4, :],
            )

    fire(0, 0)
    fire(1, 1)

    def step(it, carry):
        chunk = it * 2
        for b in range(2):
            cc = chunk + b
            drain(b)
            scatter(cc, b)

            @pl.when(cc + 2 < N_CHUNKS)
            def _():
                fire(cc + 2, b)

        return carry

    lax.fori_loop(0, N_CHUNKS // 2, step, 0)
    drain(0)
    scatter(N_CHUNKS - 1, 0)


RB = 4096
NBB = B1 // (4 * RB)


def _tr_body(x_ref, o_ref):
    xt = jnp.transpose(x_ref[...])
    for q in range(4):
        for blk in range(RB // 128):
            o_ref[0, :, blk * 512 + q * 128:blk * 512 + (q + 1) * 128] = (
                xt[32 * q:32 * q + 32, blk * 128:(blk + 1) * 128]
            )


C0 = 16384
NPB = -(-NUM_EMB // C0)


def _pack_body(x_ref, o_ref):
    o_ref[:, 0:DIM] = jnp.transpose(x_ref[...])


@jax.jit
def _emb_lookup(idx_flat, w_t):
    w_pad = pl.pallas_call(
        _pack_body,
        grid=(NPB,),
        in_specs=[pl.BlockSpec((DIM, C0), lambda i: (0, i))],
        out_specs=pl.BlockSpec((C0, 4 * DIM), lambda i: (i, 0)),
        out_shape=jax.ShapeDtypeStruct((NUM_EMB, 4 * DIM), jnp.float32),
    )(w_t)
    weight = w_pad.reshape(4 * NUM_EMB, DIM)
    mesh = plsc.VectorSubcoreMesh(
        core_axis_name="c", subcore_axis_name="s", num_cores=NC, num_subcores=NS
    )
    gather = pl.kernel(
        _gather_body,
        out_type=jax.ShapeDtypeStruct((N_BLOCKS, IDX_PER_GROUP, 4, DIM), jnp.float32),
        mesh=mesh,
        scratch_types=[
            pltpu.VMEM((GROUPS_PER_W, IDX_PER_GROUP), jnp.int32),
            pltpu.VMEM((CHUNK_ROWS, DIM), jnp.float32),
            pltpu.VMEM((CHUNK_ROWS, DIM), jnp.float32),
            pltpu.SemaphoreType.DMA,
            pltpu.SemaphoreType.DMA,
        ],
        compiler_params=pltpu.CompilerParams(use_tc_tiling_on_sc=False),
    )
    rows = gather(idx_flat, weight)

    rows_v = rows.reshape(B_TOTAL * DIM // 128, 128)

    out_t = pl.pallas_call(
        _tr_body,
        grid=(S, NBB),
        in_specs=[
            pl.BlockSpec((RB, 128), lambda si, bi: (si * NBB + bi, 0)),
        ],
        out_specs=pl.BlockSpec((1, DIM, 4 * RB), lambda si, bi: (si, 0, bi)),
        out_shape=jax.ShapeDtypeStruct((S, DIM, B1), jnp.float32),
    )(rows_v)

    return jnp.transpose(out_t, (2, 0, 1))


def kernel(input_, weight):
    idx_flat = (
        (input_.astype(jnp.int32) * 4).T
        .reshape(B_TOTAL // IDX_PER_GROUP, IDX_PER_GROUP)
    )
    return _emb_lookup(idx_flat, weight.T)
